# baseline (speedup 1.0000x reference)
import jax
import jax.numpy as jnp
from jax.experimental import pallas as pl


_B, _T, _C = 32, 4096, 128


_TBLK = 1024


def _decomp_body(x_ref, season_ref, trend_ref):
    b = pl.program_id(0)
    xv = x_ref[...]
    zero = jnp.zeros_like(xv)
    first = b == 0
    season_ref[...] = jnp.where(first, zero, xv)
    trend_ref[...] = jnp.where(first, xv, zero)


def kernel(x):
    out_shape = jax.ShapeDtypeStruct((_B, _T, _C), jnp.float32)
    spec = pl.BlockSpec((1, _TBLK, _C), lambda b, t: (b, t, 0))
    season, trend = pl.pallas_call(
        _decomp_body,
        grid=(_B, _T // _TBLK),
        in_specs=[spec],
        out_specs=[spec, spec],
        out_shape=[out_shape, out_shape],
    )(x)
    return (season, trend)


# speedup vs baseline: 1.7227x; 1.7227x over previous
import jax
import jax.numpy as jnp
from jax.experimental import pallas as pl
from jax.experimental.pallas import tpu as pltpu


_B, _T, _C = 32, 4096, 128


_BBLK = 2


def _decomp_body(x_ref, season_ref, trend_ref):
    b = pl.program_id(0)
    xv = x_ref[...]
    zero = jnp.zeros_like(xv)
    bidx = jax.lax.broadcasted_iota(jnp.int32, xv.shape, 0) + b * _BBLK
    first = bidx == 0
    season_ref[...] = jnp.where(first, zero, xv)
    trend_ref[...] = jnp.where(first, xv, zero)


def kernel(x):
    out_shape = jax.ShapeDtypeStruct((_B, _T, _C), jnp.float32)
    spec = pl.BlockSpec((_BBLK, _T, _C), lambda b: (b, 0, 0))
    season, trend = pl.pallas_call(
        _decomp_body,
        grid=(_B // _BBLK,),
        in_specs=[spec],
        out_specs=[spec, spec],
        out_shape=[out_shape, out_shape],
        compiler_params=pltpu.CompilerParams(
            dimension_semantics=("parallel",)),
    )(x)
    return (season, trend)
